# Initial kernel scaffold; baseline (speedup 1.0000x reference)
import jax
import jax.numpy as jnp
from jax.experimental import pallas as pl


def kernel(users, items, neg_items, U_mlp, U_mf, V_mlp, V_mf, W1, b1, W2, b2):
    raise NotImplementedError("write your pallas kernel here")


# speedup vs baseline: 1.8759x; 1.8759x over previous
import functools

import jax
import jax.numpy as jnp
from jax import lax
from jax.experimental import pallas as pl
from jax.experimental.pallas import tpu as pltpu
from jax.experimental.pallas import tpu_sc as plsc

B = 16384
NEG = 4
EMB = 32
NC = 2
NS = 16
NW = NC * NS
EPW = B // NW
CH = 128
POS_CH = EPW // CH
NEG_CH = EPW * NEG // CH


def _sc_gather_body(u2, i2, n2, u_mlp_t, u_mf_t, v_mlp_t, v_mf_t,
                    g_umlp, g_umf, g_vmlp, g_vmf, g_nmlp, g_nmf,
                    uv, iv, nv, r0, r1, s0, s1):
    wid = lax.axis_index("s") * NC + lax.axis_index("c")

    pltpu.sync_copy(u2.at[pl.ds(wid * POS_CH, POS_CH)], uv)
    pltpu.sync_copy(i2.at[pl.ds(wid * POS_CH, POS_CH)], iv)
    pltpu.sync_copy(n2.at[pl.ds(wid * NEG_CH, NEG_CH)], nv)

    rows = (r0, r1)
    sems = (s0, s1)

    def gather_group(table, idxv, out, nchunks, out_base):
        cps = [None, None]
        for j in range(nchunks):
            s = j % 2
            cps[s] = pltpu.async_copy(table.at[idxv.at[j]], rows[s], sems[s])
            if j >= 1:
                sp = (j - 1) % 2
                cps[sp].wait()
                pltpu.sync_copy(rows[sp],
                                out.at[pl.ds(out_base + (j - 1) * CH, CH)])
        sp = (nchunks - 1) % 2
        cps[sp].wait()
        pltpu.sync_copy(rows[sp],
                        out.at[pl.ds(out_base + (nchunks - 1) * CH, CH)])

    def gather_group_loop(table, idxv, out, nchunks, out_base):
        @pl.loop(0, nchunks, step=2)
        def _(j):
            c0 = pltpu.async_copy(table.at[idxv.at[j]], r0, s0)
            c1 = pltpu.async_copy(table.at[idxv.at[j + 1]], r1, s1)
            c0.wait()
            pltpu.sync_copy(r0, out.at[pl.ds(out_base + j * CH, CH)])
            c1.wait()
            pltpu.sync_copy(r1, out.at[pl.ds(out_base + (j + 1) * CH, CH)])

    pos_base = wid * EPW
    neg_base = wid * EPW * NEG
    gather_group(u_mlp_t, uv, g_umlp, POS_CH, pos_base)
    gather_group(u_mf_t, uv, g_umf, POS_CH, pos_base)
    gather_group(v_mlp_t, iv, g_vmlp, POS_CH, pos_base)
    gather_group(v_mf_t, iv, g_vmf, POS_CH, pos_base)
    gather_group_loop(v_mlp_t, nv, g_nmlp, NEG_CH, neg_base)
    gather_group_loop(v_mf_t, nv, g_nmf, NEG_CH, neg_base)


def _sc_gather(users, items, neg_items, U_mlp, U_mf, V_mlp, V_mf):
    u2 = users.reshape(B // CH, CH)
    i2 = items.reshape(B // CH, CH)
    n2 = neg_items.reshape(B * NEG // CH, CH)
    mesh = plsc.VectorSubcoreMesh(core_axis_name="c", subcore_axis_name="s",
                                  num_cores=NC, num_subcores=NS)
    f32 = jnp.float32
    run = pl.kernel(
        _sc_gather_body,
        out_type=[
            jax.ShapeDtypeStruct((B, EMB), f32),
            jax.ShapeDtypeStruct((B, EMB), f32),
            jax.ShapeDtypeStruct((B, EMB), f32),
            jax.ShapeDtypeStruct((B, EMB), f32),
            jax.ShapeDtypeStruct((B * NEG, EMB), f32),
            jax.ShapeDtypeStruct((B * NEG, EMB), f32),
        ],
        mesh=mesh,
        compiler_params=pltpu.CompilerParams(use_tc_tiling_on_sc=False),
        scratch_types=[
            pltpu.VMEM((POS_CH, CH), jnp.int32),
            pltpu.VMEM((POS_CH, CH), jnp.int32),
            pltpu.VMEM((NEG_CH, CH), jnp.int32),
            pltpu.VMEM((CH, EMB), f32),
            pltpu.VMEM((CH, EMB), f32),
            pltpu.SemaphoreType.DMA,
            pltpu.SemaphoreType.DMA,
        ],
    )
    return run(u2, i2, n2, U_mlp, U_mf, V_mlp, V_mf)


BT = 2048


def _tc_body(umlp, vmlp, umf, vmf, nmlp, nmf,
             w1a4, w1b, w1bd, w2, w2bd, b1r, b1t, b2r, b2t,
             o_pos, o_mf, o_nmlp, o_nmf):
    f32 = jnp.float32
    u_mf = umf[...]
    o_mf[...] = u_mf * vmf[...]
    u4 = jnp.concatenate([u_mf, u_mf, u_mf, u_mf], axis=1)
    o_nmf[...] = u4 * nmf[...]
    tu4 = jnp.dot(umlp[...], w1a4[...], preferred_element_type=f32)
    h = jnp.maximum(
        tu4[:, :EMB] + jnp.dot(vmlp[...], w1b[...], preferred_element_type=f32)
        + b1r[...], 0.0)
    o_pos[...] = jnp.maximum(
        jnp.dot(h, w2[...], preferred_element_type=f32) + b2r[...], 0.0)
    hn = jnp.maximum(
        tu4 + jnp.dot(nmlp[...], w1bd[...], preferred_element_type=f32)
        + b1t[...], 0.0)
    o_nmlp[...] = jnp.maximum(
        jnp.dot(hn, w2bd[...], preferred_element_type=f32) + b2t[...], 0.0)


def _tc_mlp(g_umlp, g_vmlp, g_umf, g_vmf, g_nmlp, g_nmf, W1, b1, W2, b2):
    f32 = jnp.float32
    nmlp = g_nmlp.reshape(B, NEG * EMB)
    nmf = g_nmf.reshape(B, NEG * EMB)
    W1a = W1[:EMB]
    W1b = W1[EMB:]
    eye4 = jnp.eye(NEG, dtype=f32)
    w1a4 = jnp.concatenate([W1a] * NEG, axis=1)
    w1bd = jnp.kron(eye4, W1b)
    w2bd = jnp.kron(eye4, W2)
    b1r = b1.reshape(1, EMB)
    b1t = jnp.tile(b1, NEG).reshape(1, NEG * EMB)
    b2r = b2.reshape(1, EMB // 2)
    b2t = jnp.tile(b2, NEG).reshape(1, NEG * EMB // 2)

    grid = (B // BT,)
    bspec = lambda shape: pl.BlockSpec(shape, lambda i: (i, 0))
    wspec = lambda shape: pl.BlockSpec(shape, lambda i: (0, 0))
    outs = pl.pallas_call(
        _tc_body,
        grid=grid,
        in_specs=[
            bspec((BT, EMB)), bspec((BT, EMB)), bspec((BT, EMB)),
            bspec((BT, EMB)), bspec((BT, NEG * EMB)), bspec((BT, NEG * EMB)),
            wspec((EMB, NEG * EMB)), wspec((EMB, EMB)),
            wspec((NEG * EMB, NEG * EMB)), wspec((EMB, EMB // 2)),
            wspec((NEG * EMB, NEG * EMB // 2)),
            wspec((1, EMB)), wspec((1, NEG * EMB)),
            wspec((1, EMB // 2)), wspec((1, NEG * EMB // 2)),
        ],
        out_specs=[
            bspec((BT, EMB // 2)), bspec((BT, EMB)),
            bspec((BT, NEG * EMB // 2)), bspec((BT, NEG * EMB)),
        ],
        out_shape=[
            jax.ShapeDtypeStruct((B, EMB // 2), f32),
            jax.ShapeDtypeStruct((B, EMB), f32),
            jax.ShapeDtypeStruct((B, NEG * EMB // 2), f32),
            jax.ShapeDtypeStruct((B, NEG * EMB), f32),
        ],
    )(g_umlp, g_vmlp, g_umf, g_vmf, nmlp, nmf,
      w1a4, W1b, w1bd, W2, w2bd, b1r, b1t, b2r, b2t)
    return outs


def kernel(users, items, neg_items, U_mlp, U_mf, V_mlp, V_mf, W1, b1, W2, b2):
    g_umlp, g_umf, g_vmlp, g_vmf, g_nmlp, g_nmf = _sc_gather(
        users, items, neg_items.reshape(-1), U_mlp, U_mf, V_mlp, V_mf)
    o_pos, o_mf, o_nmlp, o_nmf = _tc_mlp(
        g_umlp, g_vmlp, g_umf, g_vmf, g_nmlp, g_nmf, W1, b1, W2, b2)
    return (o_pos, o_mf,
            o_nmlp.reshape(B, NEG, EMB // 2),
            o_nmf.reshape(B, NEG, EMB))


# speedup vs baseline: 6.9843x; 3.7231x over previous
import functools

import jax
import jax.numpy as jnp
from jax import lax
from jax.experimental import pallas as pl
from jax.experimental.pallas import tpu as pltpu
from jax.experimental.pallas import tpu_sc as plsc

B = 16384
NEG = 4
EMB = 32
NU = 100000
NV = 1000000
NC = 2
NS = 16
NW = NC * NS
EPW = B // NW
NH = EPW * NEG // 2
CB = 16384


QB4 = CB // 4
LOG_CB = CB.bit_length() - 1
LOG_Q = QB4.bit_length() - 1


def _transpose_body(xa_ref, xb_ref, wa_ref, wb_ref):
    for x_ref, w_ref in ((xa_ref, wa_ref), (xb_ref, wb_ref)):
        x = x_ref[...]
        z = jnp.concatenate(
            [x[:, a * QB4:(a + 1) * QB4] for a in range(4)], axis=0)
        w_ref[...] = jnp.swapaxes(z, 0, 1)


def _to_rowmajor(ta, tb, n_rows):
    nb = (n_rows + CB - 1) // CB
    wa, wb = pl.pallas_call(
        _transpose_body,
        grid=(nb,),
        in_specs=[pl.BlockSpec((EMB, CB), lambda i: (0, i)),
                  pl.BlockSpec((EMB, CB), lambda i: (0, i))],
        out_specs=[pl.BlockSpec((QB4, 4 * EMB), lambda i: (i, 0)),
                   pl.BlockSpec((QB4, 4 * EMB), lambda i: (i, 0))],
        out_shape=[jax.ShapeDtypeStruct((nb * QB4, 4 * EMB), jnp.float32),
                   jax.ShapeDtypeStruct((nb * QB4, 4 * EMB), jnp.float32)],
    )(ta, tb)
    return wa.reshape(nb * CB, EMB), wb.reshape(nb * CB, EMB)


def _sc_gather_body(users, items, neg, u_mlp_t, u_mf_t, v_mlp_t, v_mf_t,
                    g_umlp, g_umf, g_vmlp, g_vmf, g_nmlp, g_nmf,
                    iv_u, iv_i, nv, pa, pb, na, nb,
                    ga, gb, gna, gnb, wa, wb, wna, wnb):
    wid = lax.axis_index("s") * NC + lax.axis_index("c")
    pos = wid * EPW
    negb = wid * EPW * NEG

    pltpu.sync_copy(users.at[pl.ds(pos, EPW)], iv_u)
    pltpu.sync_copy(items.at[pl.ds(pos, EPW)], iv_i)
    pltpu.sync_copy(neg.at[pl.ds(negb, EPW * NEG)], nv)

    def _p(r):
        return (((r >> LOG_CB) << LOG_CB) + ((r & (QB4 - 1)) << 2)
                + ((r >> LOG_Q) & 3))

    @pl.loop(0, EPW // 16)
    def _(i):
        iv_u[pl.ds(i * 16, 16)] = _p(iv_u[pl.ds(i * 16, 16)])
        iv_i[pl.ds(i * 16, 16)] = _p(iv_i[pl.ds(i * 16, 16)])

    @pl.loop(0, EPW * NEG // 16)
    def _(i):
        nv[pl.ds(i * 16, 16)] = _p(nv[pl.ds(i * 16, 16)])

    c_na = pltpu.async_copy(v_mlp_t.at[nv.at[pl.ds(0, NH)]], na, gna)
    c_nb = pltpu.async_copy(v_mlp_t.at[nv.at[pl.ds(NH, NH)]], nb, gnb)
    c_pa = pltpu.async_copy(u_mlp_t.at[iv_u], pa, ga)
    c_pb = pltpu.async_copy(u_mf_t.at[iv_u], pb, gb)

    c_na.wait()
    w_na = pltpu.async_copy(na, g_nmlp.at[pl.ds(negb, NH)], wna)
    c_nb.wait()
    w_nb = pltpu.async_copy(nb, g_nmlp.at[pl.ds(negb + NH, NH)], wnb)
    c_pa.wait()
    w_pa = pltpu.async_copy(pa, g_umlp.at[pl.ds(pos, EPW)], wa)
    c_pb.wait()
    w_pb = pltpu.async_copy(pb, g_umf.at[pl.ds(pos, EPW)], wb)

    w_na.wait()
    c_na = pltpu.async_copy(v_mf_t.at[nv.at[pl.ds(0, NH)]], na, gna)
    w_nb.wait()
    c_nb = pltpu.async_copy(v_mf_t.at[nv.at[pl.ds(NH, NH)]], nb, gnb)
    w_pa.wait()
    c_pa = pltpu.async_copy(v_mlp_t.at[iv_i], pa, ga)
    w_pb.wait()
    c_pb = pltpu.async_copy(v_mf_t.at[iv_i], pb, gb)

    c_na.wait()
    w_na = pltpu.async_copy(na, g_nmf.at[pl.ds(negb, NH)], wna)
    c_nb.wait()
    w_nb = pltpu.async_copy(nb, g_nmf.at[pl.ds(negb + NH, NH)], wnb)
    c_pa.wait()
    w_pa = pltpu.async_copy(pa, g_vmlp.at[pl.ds(pos, EPW)], wa)
    c_pb.wait()
    w_pb = pltpu.async_copy(pb, g_vmf.at[pl.ds(pos, EPW)], wb)

    w_na.wait()
    w_nb.wait()
    w_pa.wait()
    w_pb.wait()


def _sc_gather(users, items, neg_flat, u_mlp_t, u_mf_t, v_mlp_t, v_mf_t):
    mesh = plsc.VectorSubcoreMesh(core_axis_name="c", subcore_axis_name="s",
                                  num_cores=NC, num_subcores=NS)
    f32 = jnp.float32
    run = pl.kernel(
        _sc_gather_body,
        out_type=[
            jax.ShapeDtypeStruct((B, EMB), f32),
            jax.ShapeDtypeStruct((B, EMB), f32),
            jax.ShapeDtypeStruct((B, EMB), f32),
            jax.ShapeDtypeStruct((B, EMB), f32),
            jax.ShapeDtypeStruct((B * NEG, EMB), f32),
            jax.ShapeDtypeStruct((B * NEG, EMB), f32),
        ],
        mesh=mesh,
        compiler_params=pltpu.CompilerParams(use_tc_tiling_on_sc=False),
        scratch_types=[
            pltpu.VMEM((EPW,), jnp.int32),
            pltpu.VMEM((EPW,), jnp.int32),
            pltpu.VMEM((EPW * NEG,), jnp.int32),
            pltpu.VMEM((EPW, EMB), f32),
            pltpu.VMEM((EPW, EMB), f32),
            pltpu.VMEM((NH, EMB), f32),
            pltpu.VMEM((NH, EMB), f32),
            pltpu.SemaphoreType.DMA,
            pltpu.SemaphoreType.DMA,
            pltpu.SemaphoreType.DMA,
            pltpu.SemaphoreType.DMA,
            pltpu.SemaphoreType.DMA,
            pltpu.SemaphoreType.DMA,
            pltpu.SemaphoreType.DMA,
            pltpu.SemaphoreType.DMA,
        ],
    )
    return run(users, items, neg_flat, u_mlp_t, u_mf_t, v_mlp_t, v_mf_t)


BT = 2048


def _tc_body(umlp, vmlp, umf, vmf, nmlp, nmf,
             w1a4, w1b, w1bd, w2, w2bd, b1r, b1t, b2r, b2t,
             o_pos, o_mf, o_nmlp, o_nmf):
    f32 = jnp.float32
    u_mf = umf[...]
    o_mf[...] = jnp.swapaxes(u_mf * vmf[...], 0, 1)
    u4 = jnp.concatenate([u_mf, u_mf, u_mf, u_mf], axis=1)
    o_nmf[...] = jnp.swapaxes(u4 * nmf[...], 0, 1)
    tu4 = jnp.dot(umlp[...], w1a4[...], preferred_element_type=f32)
    h = jnp.maximum(
        tu4[:, :EMB] + jnp.dot(vmlp[...], w1b[...], preferred_element_type=f32)
        + b1r[...], 0.0)
    o_pos[...] = jnp.swapaxes(jnp.maximum(
        jnp.dot(h, w2[...], preferred_element_type=f32) + b2r[...], 0.0), 0, 1)
    hn = jnp.maximum(
        tu4 + jnp.dot(nmlp[...], w1bd[...], preferred_element_type=f32)
        + b1t[...], 0.0)
    o_nmlp[...] = jnp.swapaxes(jnp.maximum(
        jnp.dot(hn, w2bd[...], preferred_element_type=f32) + b2t[...], 0.0),
        0, 1)


def _tc_mlp(g_umlp, g_vmlp, g_umf, g_vmf, g_nmlp, g_nmf, W1, b1, W2, b2):
    f32 = jnp.float32
    nmlp = g_nmlp.reshape(B, NEG * EMB)
    nmf = g_nmf.reshape(B, NEG * EMB)
    W1a = W1[:EMB]
    W1b = W1[EMB:]
    eye4 = jnp.eye(NEG, dtype=f32)
    w1a4 = jnp.concatenate([W1a] * NEG, axis=1)
    w1bd = jnp.kron(eye4, W1b)
    w2bd = jnp.kron(eye4, W2)
    b1r = b1.reshape(1, EMB)
    b1t = jnp.tile(b1, NEG).reshape(1, NEG * EMB)
    b2r = b2.reshape(1, EMB // 2)
    b2t = jnp.tile(b2, NEG).reshape(1, NEG * EMB // 2)

    grid = (B // BT,)
    bspec = lambda shape: pl.BlockSpec(shape, lambda i: (i, 0))
    tspec = lambda shape: pl.BlockSpec(shape, lambda i: (0, i))
    wspec = lambda shape: pl.BlockSpec(shape, lambda i: (0, 0))
    outs = pl.pallas_call(
        _tc_body,
        grid=grid,
        in_specs=[
            bspec((BT, EMB)), bspec((BT, EMB)), bspec((BT, EMB)),
            bspec((BT, EMB)), bspec((BT, NEG * EMB)), bspec((BT, NEG * EMB)),
            wspec((EMB, NEG * EMB)), wspec((EMB, EMB)),
            wspec((NEG * EMB, NEG * EMB)), wspec((EMB, EMB // 2)),
            wspec((NEG * EMB, NEG * EMB // 2)),
            wspec((1, EMB)), wspec((1, NEG * EMB)),
            wspec((1, EMB // 2)), wspec((1, NEG * EMB // 2)),
        ],
        out_specs=[
            tspec((EMB // 2, BT)), tspec((EMB, BT)),
            tspec((NEG * EMB // 2, BT)), tspec((NEG * EMB, BT)),
        ],
        out_shape=[
            jax.ShapeDtypeStruct((EMB // 2, B), f32),
            jax.ShapeDtypeStruct((EMB, B), f32),
            jax.ShapeDtypeStruct((NEG * EMB // 2, B), f32),
            jax.ShapeDtypeStruct((NEG * EMB, B), f32),
        ],
    )(g_umlp, g_vmlp, g_umf, g_vmf, nmlp, nmf,
      w1a4, W1b, w1bd, W2, w2bd, b1r, b1t, b2r, b2t)
    return outs


def kernel(users, items, neg_items, U_mlp, U_mf, V_mlp, V_mf, W1, b1, W2, b2):
    u_mlp_t, u_mf_t = _to_rowmajor(U_mlp.T, U_mf.T, NU)
    v_mlp_t, v_mf_t = _to_rowmajor(V_mlp.T, V_mf.T, NV)
    g_umlp, g_umf, g_vmlp, g_vmf, g_nmlp, g_nmf = _sc_gather(
        users, items, neg_items.reshape(-1),
        u_mlp_t, u_mf_t, v_mlp_t, v_mf_t)
    o_pos, o_mf, o_nmlp, o_nmf = _tc_mlp(
        g_umlp, g_vmlp, g_umf, g_vmf, g_nmlp, g_nmf, W1, b1, W2, b2)
    return (o_pos.T, o_mf.T,
            o_nmlp.T.reshape(B, NEG, EMB // 2),
            o_nmf.T.reshape(B, NEG, EMB))


# speedup vs baseline: 7.0789x; 1.0135x over previous
import functools

import jax
import jax.numpy as jnp
from jax import lax
from jax.experimental import pallas as pl
from jax.experimental.pallas import tpu as pltpu
from jax.experimental.pallas import tpu_sc as plsc

B = 16384
NEG = 4
EMB = 32
NU = 100000
NV = 1000000
NC = 2
NS = 16
NW = NC * NS
EPW = B // NW
NH = EPW * NEG // 2
CB = 32768


QB4 = CB // 4
LOG_CB = CB.bit_length() - 1
LOG_Q = QB4.bit_length() - 1


def _transpose_body(xa_ref, xb_ref, wa_ref, wb_ref):
    for x_ref, w_ref in ((xa_ref, wa_ref), (xb_ref, wb_ref)):
        x = x_ref[...]
        z = jnp.concatenate(
            [x[:, a * QB4:(a + 1) * QB4] for a in range(4)], axis=0)
        w_ref[...] = jnp.swapaxes(z, 0, 1)


def _to_rowmajor(ta, tb, n_rows):
    nb = (n_rows + CB - 1) // CB
    wa, wb = pl.pallas_call(
        _transpose_body,
        grid=(nb,),
        in_specs=[pl.BlockSpec((EMB, CB), lambda i: (0, i)),
                  pl.BlockSpec((EMB, CB), lambda i: (0, i))],
        out_specs=[pl.BlockSpec((QB4, 4 * EMB), lambda i: (i, 0)),
                   pl.BlockSpec((QB4, 4 * EMB), lambda i: (i, 0))],
        out_shape=[jax.ShapeDtypeStruct((nb * QB4, 4 * EMB), jnp.float32),
                   jax.ShapeDtypeStruct((nb * QB4, 4 * EMB), jnp.float32)],
    )(ta, tb)
    return wa.reshape(nb * CB, EMB), wb.reshape(nb * CB, EMB)


def _sc_gather_body(users, items, neg, u_mlp_t, u_mf_t, v_mlp_t, v_mf_t,
                    g_umlp, g_umf, g_vmlp, g_vmf, g_nmlp, g_nmf,
                    iv_u, iv_i, nv, pa, pb, na, nb,
                    ga, gb, gna, gnb, wa, wb, wna, wnb):
    wid = lax.axis_index("s") * NC + lax.axis_index("c")
    pos = wid * EPW
    negb = wid * EPW * NEG

    pltpu.sync_copy(users.at[pl.ds(pos, EPW)], iv_u)
    pltpu.sync_copy(items.at[pl.ds(pos, EPW)], iv_i)
    pltpu.sync_copy(neg.at[pl.ds(negb, EPW * NEG)], nv)

    def _p(r):
        return (((r >> LOG_CB) << LOG_CB) + ((r & (QB4 - 1)) << 2)
                + ((r >> LOG_Q) & 3))

    @pl.loop(0, EPW // 16)
    def _(i):
        iv_u[pl.ds(i * 16, 16)] = _p(iv_u[pl.ds(i * 16, 16)])
        iv_i[pl.ds(i * 16, 16)] = _p(iv_i[pl.ds(i * 16, 16)])

    @pl.loop(0, EPW * NEG // 16)
    def _(i):
        nv[pl.ds(i * 16, 16)] = _p(nv[pl.ds(i * 16, 16)])

    c_na = pltpu.async_copy(v_mlp_t.at[nv.at[pl.ds(0, NH)]], na, gna)
    c_nb = pltpu.async_copy(v_mlp_t.at[nv.at[pl.ds(NH, NH)]], nb, gnb)
    c_pa = pltpu.async_copy(u_mlp_t.at[iv_u], pa, ga)
    c_pb = pltpu.async_copy(u_mf_t.at[iv_u], pb, gb)

    c_na.wait()
    w_na = pltpu.async_copy(na, g_nmlp.at[pl.ds(negb, NH)], wna)
    c_nb.wait()
    w_nb = pltpu.async_copy(nb, g_nmlp.at[pl.ds(negb + NH, NH)], wnb)
    c_pa.wait()
    w_pa = pltpu.async_copy(pa, g_umlp.at[pl.ds(pos, EPW)], wa)
    c_pb.wait()
    w_pb = pltpu.async_copy(pb, g_umf.at[pl.ds(pos, EPW)], wb)

    w_na.wait()
    c_na = pltpu.async_copy(v_mf_t.at[nv.at[pl.ds(0, NH)]], na, gna)
    w_nb.wait()
    c_nb = pltpu.async_copy(v_mf_t.at[nv.at[pl.ds(NH, NH)]], nb, gnb)
    w_pa.wait()
    c_pa = pltpu.async_copy(v_mlp_t.at[iv_i], pa, ga)
    w_pb.wait()
    c_pb = pltpu.async_copy(v_mf_t.at[iv_i], pb, gb)

    c_na.wait()
    w_na = pltpu.async_copy(na, g_nmf.at[pl.ds(negb, NH)], wna)
    c_nb.wait()
    w_nb = pltpu.async_copy(nb, g_nmf.at[pl.ds(negb + NH, NH)], wnb)
    c_pa.wait()
    w_pa = pltpu.async_copy(pa, g_vmlp.at[pl.ds(pos, EPW)], wa)
    c_pb.wait()
    w_pb = pltpu.async_copy(pb, g_vmf.at[pl.ds(pos, EPW)], wb)

    w_na.wait()
    w_nb.wait()
    w_pa.wait()
    w_pb.wait()


def _sc_gather(users, items, neg_flat, u_mlp_t, u_mf_t, v_mlp_t, v_mf_t):
    mesh = plsc.VectorSubcoreMesh(core_axis_name="c", subcore_axis_name="s",
                                  num_cores=NC, num_subcores=NS)
    f32 = jnp.float32
    run = pl.kernel(
        _sc_gather_body,
        out_type=[
            jax.ShapeDtypeStruct((B, EMB), f32),
            jax.ShapeDtypeStruct((B, EMB), f32),
            jax.ShapeDtypeStruct((B, EMB), f32),
            jax.ShapeDtypeStruct((B, EMB), f32),
            jax.ShapeDtypeStruct((B * NEG, EMB), f32),
            jax.ShapeDtypeStruct((B * NEG, EMB), f32),
        ],
        mesh=mesh,
        compiler_params=pltpu.CompilerParams(use_tc_tiling_on_sc=False),
        scratch_types=[
            pltpu.VMEM((EPW,), jnp.int32),
            pltpu.VMEM((EPW,), jnp.int32),
            pltpu.VMEM((EPW * NEG,), jnp.int32),
            pltpu.VMEM((EPW, EMB), f32),
            pltpu.VMEM((EPW, EMB), f32),
            pltpu.VMEM((NH, EMB), f32),
            pltpu.VMEM((NH, EMB), f32),
            pltpu.SemaphoreType.DMA,
            pltpu.SemaphoreType.DMA,
            pltpu.SemaphoreType.DMA,
            pltpu.SemaphoreType.DMA,
            pltpu.SemaphoreType.DMA,
            pltpu.SemaphoreType.DMA,
            pltpu.SemaphoreType.DMA,
            pltpu.SemaphoreType.DMA,
        ],
    )
    return run(users, items, neg_flat, u_mlp_t, u_mf_t, v_mlp_t, v_mf_t)


BT = 2048


def _tc_body(umlp, vmlp, umf, vmf, nmlp, nmf,
             w1a4, w1b, w1bd, w2, w2bd, b1r, b1t, b2r, b2t,
             o_pos, o_mf, o_nmlp, o_nmf):
    f32 = jnp.float32
    u_mf = umf[...]
    o_mf[...] = jnp.swapaxes(u_mf * vmf[...], 0, 1)
    u4 = jnp.concatenate([u_mf, u_mf, u_mf, u_mf], axis=1)
    o_nmf[...] = jnp.swapaxes(u4 * nmf[...], 0, 1)
    tu4 = jnp.dot(umlp[...], w1a4[...], preferred_element_type=f32)
    h = jnp.maximum(
        tu4[:, :EMB] + jnp.dot(vmlp[...], w1b[...], preferred_element_type=f32)
        + b1r[...], 0.0)
    o_pos[...] = jnp.swapaxes(jnp.maximum(
        jnp.dot(h, w2[...], preferred_element_type=f32) + b2r[...], 0.0), 0, 1)
    hn = jnp.maximum(
        tu4 + jnp.dot(nmlp[...], w1bd[...], preferred_element_type=f32)
        + b1t[...], 0.0)
    o_nmlp[...] = jnp.swapaxes(jnp.maximum(
        jnp.dot(hn, w2bd[...], preferred_element_type=f32) + b2t[...], 0.0),
        0, 1)


def _tc_mlp(g_umlp, g_vmlp, g_umf, g_vmf, g_nmlp, g_nmf, W1, b1, W2, b2):
    f32 = jnp.float32
    nmlp = g_nmlp.reshape(B, NEG * EMB)
    nmf = g_nmf.reshape(B, NEG * EMB)
    W1a = W1[:EMB]
    W1b = W1[EMB:]
    eye4 = jnp.eye(NEG, dtype=f32)
    w1a4 = jnp.concatenate([W1a] * NEG, axis=1)
    w1bd = jnp.kron(eye4, W1b)
    w2bd = jnp.kron(eye4, W2)
    b1r = b1.reshape(1, EMB)
    b1t = jnp.tile(b1, NEG).reshape(1, NEG * EMB)
    b2r = b2.reshape(1, EMB // 2)
    b2t = jnp.tile(b2, NEG).reshape(1, NEG * EMB // 2)

    grid = (B // BT,)
    bspec = lambda shape: pl.BlockSpec(shape, lambda i: (i, 0))
    tspec = lambda shape: pl.BlockSpec(shape, lambda i: (0, i))
    wspec = lambda shape: pl.BlockSpec(shape, lambda i: (0, 0))
    outs = pl.pallas_call(
        _tc_body,
        grid=grid,
        in_specs=[
            bspec((BT, EMB)), bspec((BT, EMB)), bspec((BT, EMB)),
            bspec((BT, EMB)), bspec((BT, NEG * EMB)), bspec((BT, NEG * EMB)),
            wspec((EMB, NEG * EMB)), wspec((EMB, EMB)),
            wspec((NEG * EMB, NEG * EMB)), wspec((EMB, EMB // 2)),
            wspec((NEG * EMB, NEG * EMB // 2)),
            wspec((1, EMB)), wspec((1, NEG * EMB)),
            wspec((1, EMB // 2)), wspec((1, NEG * EMB // 2)),
        ],
        out_specs=[
            tspec((EMB // 2, BT)), tspec((EMB, BT)),
            tspec((NEG * EMB // 2, BT)), tspec((NEG * EMB, BT)),
        ],
        out_shape=[
            jax.ShapeDtypeStruct((EMB // 2, B), f32),
            jax.ShapeDtypeStruct((EMB, B), f32),
            jax.ShapeDtypeStruct((NEG * EMB // 2, B), f32),
            jax.ShapeDtypeStruct((NEG * EMB, B), f32),
        ],
    )(g_umlp, g_vmlp, g_umf, g_vmf, nmlp, nmf,
      w1a4, W1b, w1bd, W2, w2bd, b1r, b1t, b2r, b2t)
    return outs


def kernel(users, items, neg_items, U_mlp, U_mf, V_mlp, V_mf, W1, b1, W2, b2):
    u_mlp_t, u_mf_t = _to_rowmajor(U_mlp.T, U_mf.T, NU)
    v_mlp_t, v_mf_t = _to_rowmajor(V_mlp.T, V_mf.T, NV)
    g_umlp, g_umf, g_vmlp, g_vmf, g_nmlp, g_nmf = _sc_gather(
        users, items, neg_items.reshape(-1),
        u_mlp_t, u_mf_t, v_mlp_t, v_mf_t)
    o_pos, o_mf, o_nmlp, o_nmf = _tc_mlp(
        g_umlp, g_vmlp, g_umf, g_vmf, g_nmlp, g_nmf, W1, b1, W2, b2)
    return (o_pos.T, o_mf.T,
            o_nmlp.T.reshape(B, NEG, EMB // 2),
            o_nmf.T.reshape(B, NEG, EMB))


# speedup vs baseline: 7.0892x; 1.0015x over previous
import functools

import jax
import jax.numpy as jnp
from jax import lax
from jax.experimental import pallas as pl
from jax.experimental.pallas import tpu as pltpu
from jax.experimental.pallas import tpu_sc as plsc

B = 16384
NEG = 4
EMB = 32
NU = 100000
NV = 1000000
NC = 2
NS = 16
NW = NC * NS
EPW = B // NW
NH = EPW * NEG // 2
CB = 32768


QB4 = CB // 4
LOG_CB = CB.bit_length() - 1
LOG_Q = QB4.bit_length() - 1


def _transpose_body(xa_ref, xb_ref, wa_ref, wb_ref):
    for x_ref, w_ref in ((xa_ref, wa_ref), (xb_ref, wb_ref)):
        x = x_ref[...]
        z = jnp.concatenate(
            [x[:, a * QB4:(a + 1) * QB4] for a in range(4)], axis=0)
        w_ref[...] = jnp.swapaxes(z, 0, 1)


def _to_rowmajor(ta, tb, n_rows):
    nb = (n_rows + CB - 1) // CB
    wa, wb = pl.pallas_call(
        _transpose_body,
        grid=(nb,),
        in_specs=[pl.BlockSpec((EMB, CB), lambda i: (0, i)),
                  pl.BlockSpec((EMB, CB), lambda i: (0, i))],
        out_specs=[pl.BlockSpec((QB4, 4 * EMB), lambda i: (i, 0)),
                   pl.BlockSpec((QB4, 4 * EMB), lambda i: (i, 0))],
        out_shape=[jax.ShapeDtypeStruct((nb * QB4, 4 * EMB), jnp.float32),
                   jax.ShapeDtypeStruct((nb * QB4, 4 * EMB), jnp.float32)],
    )(ta, tb)
    return wa.reshape(nb * CB, EMB), wb.reshape(nb * CB, EMB)


def _sc_gather_body(users, items, neg, u_mlp_t, u_mf_t, v_mlp_t, v_mf_t,
                    g_umlp, g_umf, g_vmlp, g_vmf, g_nmlp, g_nmf,
                    iv_u, iv_i, nv, pa, pb, na, nb,
                    ga, gb, gna, gnb, wa, wb, wna, wnb):
    wid = lax.axis_index("s") * NC + lax.axis_index("c")
    pos = wid * EPW
    negb = wid * EPW * NEG

    pltpu.sync_copy(users.at[pl.ds(pos, EPW)], iv_u)
    pltpu.sync_copy(items.at[pl.ds(pos, EPW)], iv_i)
    pltpu.sync_copy(neg.at[pl.ds(negb, EPW * NEG)], nv)

    def _p(r):
        return (((r >> LOG_CB) << LOG_CB) + ((r & (QB4 - 1)) << 2)
                + ((r >> LOG_Q) & 3))

    @pl.loop(0, EPW // 16)
    def _(i):
        iv_u[pl.ds(i * 16, 16)] = _p(iv_u[pl.ds(i * 16, 16)])
        iv_i[pl.ds(i * 16, 16)] = _p(iv_i[pl.ds(i * 16, 16)])

    @pl.loop(0, EPW * NEG // 16)
    def _(i):
        nv[pl.ds(i * 16, 16)] = _p(nv[pl.ds(i * 16, 16)])

    c_na = pltpu.async_copy(v_mlp_t.at[nv.at[pl.ds(0, NH)]], na, gna)
    c_nb = pltpu.async_copy(v_mlp_t.at[nv.at[pl.ds(NH, NH)]], nb, gnb)
    c_pa = pltpu.async_copy(u_mlp_t.at[iv_u], pa, ga)
    c_pb = pltpu.async_copy(u_mf_t.at[iv_u], pb, gb)

    c_na.wait()
    w_na = pltpu.async_copy(na, g_nmlp.at[pl.ds(negb, NH)], wna)
    c_nb.wait()
    w_nb = pltpu.async_copy(nb, g_nmlp.at[pl.ds(negb + NH, NH)], wnb)
    c_pa.wait()
    w_pa = pltpu.async_copy(pa, g_umlp.at[pl.ds(pos, EPW)], wa)
    c_pb.wait()
    w_pb = pltpu.async_copy(pb, g_umf.at[pl.ds(pos, EPW)], wb)

    w_na.wait()
    c_na = pltpu.async_copy(v_mf_t.at[nv.at[pl.ds(0, NH)]], na, gna)
    w_nb.wait()
    c_nb = pltpu.async_copy(v_mf_t.at[nv.at[pl.ds(NH, NH)]], nb, gnb)
    w_pa.wait()
    c_pa = pltpu.async_copy(v_mlp_t.at[iv_i], pa, ga)
    w_pb.wait()
    c_pb = pltpu.async_copy(v_mf_t.at[iv_i], pb, gb)

    c_na.wait()
    w_na = pltpu.async_copy(na, g_nmf.at[pl.ds(negb, NH)], wna)
    c_nb.wait()
    w_nb = pltpu.async_copy(nb, g_nmf.at[pl.ds(negb + NH, NH)], wnb)
    c_pa.wait()
    w_pa = pltpu.async_copy(pa, g_vmlp.at[pl.ds(pos, EPW)], wa)
    c_pb.wait()
    w_pb = pltpu.async_copy(pb, g_vmf.at[pl.ds(pos, EPW)], wb)

    w_na.wait()
    w_nb.wait()
    w_pa.wait()
    w_pb.wait()


def _sc_gather(users, items, neg_flat, u_mlp_t, u_mf_t, v_mlp_t, v_mf_t):
    mesh = plsc.VectorSubcoreMesh(core_axis_name="c", subcore_axis_name="s",
                                  num_cores=NC, num_subcores=NS)
    f32 = jnp.float32
    run = pl.kernel(
        _sc_gather_body,
        out_type=[
            jax.ShapeDtypeStruct((B, EMB), f32),
            jax.ShapeDtypeStruct((B, EMB), f32),
            jax.ShapeDtypeStruct((B, EMB), f32),
            jax.ShapeDtypeStruct((B, EMB), f32),
            jax.ShapeDtypeStruct((B * NEG, EMB), f32),
            jax.ShapeDtypeStruct((B * NEG, EMB), f32),
        ],
        mesh=mesh,
        compiler_params=pltpu.CompilerParams(use_tc_tiling_on_sc=False),
        scratch_types=[
            pltpu.VMEM((EPW,), jnp.int32),
            pltpu.VMEM((EPW,), jnp.int32),
            pltpu.VMEM((EPW * NEG,), jnp.int32),
            pltpu.VMEM((EPW, EMB), f32),
            pltpu.VMEM((EPW, EMB), f32),
            pltpu.VMEM((NH, EMB), f32),
            pltpu.VMEM((NH, EMB), f32),
            pltpu.SemaphoreType.DMA,
            pltpu.SemaphoreType.DMA,
            pltpu.SemaphoreType.DMA,
            pltpu.SemaphoreType.DMA,
            pltpu.SemaphoreType.DMA,
            pltpu.SemaphoreType.DMA,
            pltpu.SemaphoreType.DMA,
            pltpu.SemaphoreType.DMA,
        ],
    )
    return run(users, items, neg_flat, u_mlp_t, u_mf_t, v_mlp_t, v_mf_t)


BT = 4096


def _tc_body(umlp, vmlp, umf, vmf, nmlp, nmf,
             w1a4, w1b, w1bd, w2, w2bd, b1r, b1t, b2r, b2t,
             o_pos, o_mf, o_nmlp, o_nmf):
    f32 = jnp.float32
    u_mf = umf[...]
    o_mf[...] = jnp.swapaxes(u_mf * vmf[...], 0, 1)
    u4 = jnp.concatenate([u_mf, u_mf, u_mf, u_mf], axis=1)
    o_nmf[...] = jnp.swapaxes(u4 * nmf[...], 0, 1)
    tu4 = jnp.dot(umlp[...], w1a4[...], preferred_element_type=f32)
    h = jnp.maximum(
        tu4[:, :EMB] + jnp.dot(vmlp[...], w1b[...], preferred_element_type=f32)
        + b1r[...], 0.0)
    o_pos[...] = jnp.swapaxes(jnp.maximum(
        jnp.dot(h, w2[...], preferred_element_type=f32) + b2r[...], 0.0), 0, 1)
    hn = jnp.maximum(
        tu4 + jnp.dot(nmlp[...], w1bd[...], preferred_element_type=f32)
        + b1t[...], 0.0)
    o_nmlp[...] = jnp.swapaxes(jnp.maximum(
        jnp.dot(hn, w2bd[...], preferred_element_type=f32) + b2t[...], 0.0),
        0, 1)


def _tc_mlp(g_umlp, g_vmlp, g_umf, g_vmf, g_nmlp, g_nmf, W1, b1, W2, b2):
    f32 = jnp.float32
    nmlp = g_nmlp.reshape(B, NEG * EMB)
    nmf = g_nmf.reshape(B, NEG * EMB)
    W1a = W1[:EMB]
    W1b = W1[EMB:]
    eye4 = jnp.eye(NEG, dtype=f32)
    w1a4 = jnp.concatenate([W1a] * NEG, axis=1)
    w1bd = jnp.kron(eye4, W1b)
    w2bd = jnp.kron(eye4, W2)
    b1r = b1.reshape(1, EMB)
    b1t = jnp.tile(b1, NEG).reshape(1, NEG * EMB)
    b2r = b2.reshape(1, EMB // 2)
    b2t = jnp.tile(b2, NEG).reshape(1, NEG * EMB // 2)

    grid = (B // BT,)
    bspec = lambda shape: pl.BlockSpec(shape, lambda i: (i, 0))
    tspec = lambda shape: pl.BlockSpec(shape, lambda i: (0, i))
    wspec = lambda shape: pl.BlockSpec(shape, lambda i: (0, 0))
    outs = pl.pallas_call(
        _tc_body,
        grid=grid,
        in_specs=[
            bspec((BT, EMB)), bspec((BT, EMB)), bspec((BT, EMB)),
            bspec((BT, EMB)), bspec((BT, NEG * EMB)), bspec((BT, NEG * EMB)),
            wspec((EMB, NEG * EMB)), wspec((EMB, EMB)),
            wspec((NEG * EMB, NEG * EMB)), wspec((EMB, EMB // 2)),
            wspec((NEG * EMB, NEG * EMB // 2)),
            wspec((1, EMB)), wspec((1, NEG * EMB)),
            wspec((1, EMB // 2)), wspec((1, NEG * EMB // 2)),
        ],
        out_specs=[
            tspec((EMB // 2, BT)), tspec((EMB, BT)),
            tspec((NEG * EMB // 2, BT)), tspec((NEG * EMB, BT)),
        ],
        out_shape=[
            jax.ShapeDtypeStruct((EMB // 2, B), f32),
            jax.ShapeDtypeStruct((EMB, B), f32),
            jax.ShapeDtypeStruct((NEG * EMB // 2, B), f32),
            jax.ShapeDtypeStruct((NEG * EMB, B), f32),
        ],
    )(g_umlp, g_vmlp, g_umf, g_vmf, nmlp, nmf,
      w1a4, W1b, w1bd, W2, w2bd, b1r, b1t, b2r, b2t)
    return outs


def kernel(users, items, neg_items, U_mlp, U_mf, V_mlp, V_mf, W1, b1, W2, b2):
    u_mlp_t, u_mf_t = _to_rowmajor(U_mlp.T, U_mf.T, NU)
    v_mlp_t, v_mf_t = _to_rowmajor(V_mlp.T, V_mf.T, NV)
    g_umlp, g_umf, g_vmlp, g_vmf, g_nmlp, g_nmf = _sc_gather(
        users, items, neg_items.reshape(-1),
        u_mlp_t, u_mf_t, v_mlp_t, v_mf_t)
    o_pos, o_mf, o_nmlp, o_nmf = _tc_mlp(
        g_umlp, g_vmlp, g_umf, g_vmf, g_nmlp, g_nmf, W1, b1, W2, b2)
    return (o_pos.T, o_mf.T,
            o_nmlp.T.reshape(B, NEG, EMB // 2),
            o_nmf.T.reshape(B, NEG, EMB))
